# Initial kernel scaffold; baseline (speedup 1.0000x reference)
import jax
import jax.numpy as jnp
from jax.experimental import pallas as pl


def kernel(x, edge_index, n_qubits, in_W, in_b, msg_W1, msg_b1, msg_W2, msg_b2, upd_W1, upd_b1, upd_W2, upd_b2, ln_g, ln_b, out_W1, out_b1, out_W2, out_b2):
    raise NotImplementedError("write your pallas kernel here")


# speedup vs baseline: 2.7860x; 2.7860x over previous
import functools

import jax
import jax.numpy as jnp
from jax import lax
from jax.experimental import pallas as pl
from jax.experimental.pallas import tpu as pltpu
from jax.experimental.pallas import tpu_sc as plsc

N = 10000
E = 160000
D = 256
L = 8
NQ = 8192
DH = D // 2

_NC = 2
_NS = 16
_EB = 128
_EROWS = 1280
_EPAD = _EROWS * _EB
_RPT = _EROWS // _NS
_CHJ = 8
_GW = 2
_NCH = _RPT // _CHJ
_NACC = 10112


def _mm_bias_relu_body(x_ref, w_ref, b_ref, o_ref):
    o_ref[...] = jnp.maximum(
        jnp.dot(x_ref[...], w_ref[...], preferred_element_type=jnp.float32)
        + b_ref[...], 0.0)


def _in_mlp(x, w, b):
    tm = 1000
    return pl.pallas_call(
        _mm_bias_relu_body,
        grid=(N // tm,),
        in_specs=[
            pl.BlockSpec((tm, D), lambda m: (m, 0)),
            pl.BlockSpec((D, D), lambda m: (0, 0)),
            pl.BlockSpec((1, D), lambda m: (0, 0)),
        ],
        out_specs=pl.BlockSpec((tm, D), lambda m: (m, 0)),
        out_shape=jax.ShapeDtypeStruct((N, D), jnp.float32),
    )(x, w, b.reshape(1, D))


def _msg_body(h_ref, w1_ref, b1_ref, w2_ref, b2_ref, o_ref):
    t = jnp.maximum(
        jnp.dot(h_ref[...], w1_ref[...], preferred_element_type=jnp.float32)
        + b1_ref[...], 0.0)
    m = jnp.dot(t, w2_ref[...], preferred_element_type=jnp.float32) + b2_ref[...]
    o_ref[0] = m[:, :DH]
    o_ref[1] = m[:, DH:]


def _msg_mlp(h, w1, b1, w2, b2):
    tm = 1000
    return pl.pallas_call(
        _msg_body,
        grid=(N // tm,),
        in_specs=[
            pl.BlockSpec((tm, D), lambda m: (m, 0)),
            pl.BlockSpec((D, D), lambda m: (0, 0)),
            pl.BlockSpec((1, D), lambda m: (0, 0)),
            pl.BlockSpec((D, D), lambda m: (0, 0)),
            pl.BlockSpec((1, D), lambda m: (0, 0)),
        ],
        out_specs=pl.BlockSpec((_NC, tm, DH), lambda m: (0, m, 0)),
        out_shape=jax.ShapeDtypeStruct((_NC, N, DH), jnp.float32),
    )(h, w1, b1.reshape(1, D), w2, b2.reshape(1, D))


def _upd_body(h_ref, agg_ref, w1_ref, b1_ref, w2_ref, b2_ref, g_ref, bb_ref, o_ref):
    h = h_ref[...]
    w1 = w1_ref[...]
    t = (jnp.dot(h, w1[:D], preferred_element_type=jnp.float32)
         + jnp.dot(agg_ref[0], w1[D:D + DH], preferred_element_type=jnp.float32)
         + jnp.dot(agg_ref[1], w1[D + DH:], preferred_element_type=jnp.float32)
         + b1_ref[...])
    t = jnp.maximum(t, 0.0)
    y = jnp.dot(t, w2_ref[...], preferred_element_type=jnp.float32) + b2_ref[...] + h
    mu = jnp.mean(y, axis=1, keepdims=True)
    yc = y - mu
    var = jnp.mean(yc * yc, axis=1, keepdims=True)
    o_ref[...] = jnp.maximum(
        g_ref[...] * yc / jnp.sqrt(var + 1e-5) + bb_ref[...], 0.0)


def _upd_mlp(h, agg, w1, b1, w2, b2, g, bb):
    tm = 1000
    return pl.pallas_call(
        _upd_body,
        grid=(N // tm,),
        in_specs=[
            pl.BlockSpec((tm, D), lambda m: (m, 0)),
            pl.BlockSpec((_NC, tm, DH), lambda m: (0, m, 0)),
            pl.BlockSpec((2 * D, D), lambda m: (0, 0)),
            pl.BlockSpec((1, D), lambda m: (0, 0)),
            pl.BlockSpec((D, D), lambda m: (0, 0)),
            pl.BlockSpec((1, D), lambda m: (0, 0)),
            pl.BlockSpec((1, D), lambda m: (0, 0)),
            pl.BlockSpec((1, D), lambda m: (0, 0)),
        ],
        out_specs=pl.BlockSpec((tm, D), lambda m: (m, 0)),
        out_shape=jax.ShapeDtypeStruct((N, D), jnp.float32),
    )(h, agg, w1, b1.reshape(1, D), w2, b2.reshape(1, D),
      g.reshape(1, D), bb.reshape(1, D))


def _out_body(q_ref, w1_ref, b1_ref, w2_ref, o_ref):
    t = jnp.maximum(
        jnp.dot(q_ref[...], w1_ref[...], preferred_element_type=jnp.float32)
        + b1_ref[...], 0.0)
    o_ref[...] = jnp.sum(t * w2_ref[...], axis=1, keepdims=True)


def _out_mlp(q, w1, b1, w2, b2):
    tm = 1024
    s = pl.pallas_call(
        _out_body,
        grid=(NQ // tm,),
        in_specs=[
            pl.BlockSpec((tm, D), lambda m: (m, 0)),
            pl.BlockSpec((D, D), lambda m: (0, 0)),
            pl.BlockSpec((1, D), lambda m: (0, 0)),
            pl.BlockSpec((1, D), lambda m: (0, 0)),
        ],
        out_specs=pl.BlockSpec((tm, 1), lambda m: (m, 0)),
        out_shape=jax.ShapeDtypeStruct((NQ, 1), jnp.float32),
    )(q, w1, b1.reshape(1, D), w2.reshape(1, D))
    return s + b2


@functools.lru_cache(maxsize=None)
def _sc_aggregate_fn():
    mesh = plsc.VectorSubcoreMesh(core_axis_name="c", subcore_axis_name="s")

    @functools.partial(
        pl.kernel,
        mesh=mesh,
        out_type=jax.ShapeDtypeStruct((_NC, _NACC, DH), jnp.float32),
        scratch_types=[
            pltpu.VMEM((_CHJ, _EB), jnp.int32),
            pltpu.VMEM((_CHJ, _EB), jnp.int32),
            pltpu.VMEM((_GW * _EB, DH), jnp.float32),
            pltpu.VMEM_SHARED((_NACC, DH), jnp.float32),
            pltpu.SemaphoreType.DMA,
        ],
    )
    def aggr(msg_hbm, src_hbm, dst_hbm, zero_hbm, out_hbm,
             src_v, dst_v, rows_v, acc_sh, sem):
        c = lax.axis_index("c")
        s = lax.axis_index("s")
        npt = _NACC // _NS
        pltpu.sync_copy(zero_hbm.at[pl.ds(s * npt, npt)],
                        acc_sh.at[pl.ds(s * npt, npt)])
        plsc.subcore_barrier()
        msg_c = msg_hbm.at[c]
        base = s * _RPT

        def body(k, carry):
            r0 = base + k * _CHJ
            pltpu.sync_copy(src_hbm.at[pl.ds(r0, _CHJ)], src_v)
            pltpu.sync_copy(dst_hbm.at[pl.ds(r0, _CHJ)], dst_v)
            for g in range(_CHJ // _GW):
                cps = [pltpu.make_async_copy(msg_c.at[src_v.at[g * _GW + j]],
                                             rows_v.at[pl.ds(j * _EB, _EB)],
                                             sem)
                       for j in range(_GW)]
                for cp in cps:
                    cp.start()
                for cp in cps:
                    cp.wait()
                for j in range(_GW):
                    pltpu.sync_copy(rows_v.at[pl.ds(j * _EB, _EB)],
                                    acc_sh.at[dst_v.at[g * _GW + j]], add=True)
            return carry

        lax.fori_loop(0, _NCH, body, 0)
        plsc.subcore_barrier()
        pltpu.sync_copy(acc_sh.at[pl.ds(s * npt, npt)],
                        out_hbm.at[c, pl.ds(s * npt, npt)])

    return aggr


def kernel(x, edge_index, n_qubits, in_W, in_b, msg_W1, msg_b1, msg_W2, msg_b2,
           upd_W1, upd_b1, upd_W2, upd_b2, ln_g, ln_b,
           out_W1, out_b1, out_W2, out_b2):
    pad = _EPAD - E
    src_r = jnp.concatenate(
        [edge_index[0], jnp.zeros((pad,), edge_index.dtype)]).reshape(_EROWS, _EB)
    dst_r = jnp.concatenate(
        [edge_index[1], jnp.full((pad,), N, edge_index.dtype)]).reshape(_EROWS, _EB)
    zeros = jnp.zeros((_NACC, DH), jnp.float32)
    aggr_fn = _sc_aggregate_fn()

    h = _in_mlp(x, in_W, in_b)
    for i in range(L):
        msg2 = _msg_mlp(h, msg_W1[i], msg_b1[i], msg_W2[i], msg_b2[i])
        agg2 = aggr_fn(msg2, src_r, dst_r, zeros)
        h = _upd_mlp(h, agg2, upd_W1[i], upd_b1[i], upd_W2[i], upd_b2[i],
                     ln_g[i], ln_b[i])
    q = lax.dynamic_slice_in_dim(h, n_qubits - NQ, NQ, axis=0)
    return _out_mlp(q, out_W1, out_b1, out_W2, out_b2)


# speedup vs baseline: 3.1911x; 1.1454x over previous
import functools

import jax
import jax.numpy as jnp
from jax import lax
from jax.experimental import pallas as pl
from jax.experimental.pallas import tpu as pltpu
from jax.experimental.pallas import tpu_sc as plsc

N = 10000
E = 160000
D = 256
L = 8
NQ = 8192
DH = D // 2

_NC = 2
_NS = 16
_EB = 128
_EROWS = 1280
_EPAD = _EROWS * _EB
_RPT = _EROWS // _NS
_ICH = 16
_NICH = _RPT // _ICH
_NACC = 10112


def _mm_bias_relu_body(x_ref, w_ref, b_ref, o_ref):
    o_ref[...] = jnp.maximum(
        jnp.dot(x_ref[...], w_ref[...], preferred_element_type=jnp.float32)
        + b_ref[...], 0.0)


def _in_mlp(x, w, b):
    tm = 1000
    return pl.pallas_call(
        _mm_bias_relu_body,
        grid=(N // tm,),
        in_specs=[
            pl.BlockSpec((tm, D), lambda m: (m, 0)),
            pl.BlockSpec((D, D), lambda m: (0, 0)),
            pl.BlockSpec((1, D), lambda m: (0, 0)),
        ],
        out_specs=pl.BlockSpec((tm, D), lambda m: (m, 0)),
        out_shape=jax.ShapeDtypeStruct((N, D), jnp.float32),
    )(x, w, b.reshape(1, D))


def _msg_body(h_ref, w1_ref, b1_ref, w2_ref, b2_ref, o_ref):
    t = jnp.maximum(
        jnp.dot(h_ref[...], w1_ref[...], preferred_element_type=jnp.float32)
        + b1_ref[...], 0.0)
    m = jnp.dot(t, w2_ref[...], preferred_element_type=jnp.float32) + b2_ref[...]
    o_ref[0] = m[:, :DH]
    o_ref[1] = m[:, DH:]


def _msg_mlp(h, w1, b1, w2, b2):
    tm = 1000
    return pl.pallas_call(
        _msg_body,
        grid=(N // tm,),
        in_specs=[
            pl.BlockSpec((tm, D), lambda m: (m, 0)),
            pl.BlockSpec((D, D), lambda m: (0, 0)),
            pl.BlockSpec((1, D), lambda m: (0, 0)),
            pl.BlockSpec((D, D), lambda m: (0, 0)),
            pl.BlockSpec((1, D), lambda m: (0, 0)),
        ],
        out_specs=pl.BlockSpec((_NC, tm, DH), lambda m: (0, m, 0)),
        out_shape=jax.ShapeDtypeStruct((_NC, N, DH), jnp.float32),
    )(h, w1, b1.reshape(1, D), w2, b2.reshape(1, D))


def _upd_body(h_ref, agg_ref, w1_ref, b1_ref, w2_ref, b2_ref, g_ref, bb_ref, o_ref):
    h = h_ref[...]
    w1 = w1_ref[...]
    t = (jnp.dot(h, w1[:D], preferred_element_type=jnp.float32)
         + jnp.dot(agg_ref[0], w1[D:D + DH], preferred_element_type=jnp.float32)
         + jnp.dot(agg_ref[1], w1[D + DH:], preferred_element_type=jnp.float32)
         + b1_ref[...])
    t = jnp.maximum(t, 0.0)
    y = jnp.dot(t, w2_ref[...], preferred_element_type=jnp.float32) + b2_ref[...] + h
    mu = jnp.mean(y, axis=1, keepdims=True)
    yc = y - mu
    var = jnp.mean(yc * yc, axis=1, keepdims=True)
    o_ref[...] = jnp.maximum(
        g_ref[...] * yc / jnp.sqrt(var + 1e-5) + bb_ref[...], 0.0)


def _upd_mlp(h, agg, w1, b1, w2, b2, g, bb):
    tm = 1000
    return pl.pallas_call(
        _upd_body,
        grid=(N // tm,),
        in_specs=[
            pl.BlockSpec((tm, D), lambda m: (m, 0)),
            pl.BlockSpec((_NC, tm, DH), lambda m: (0, m, 0)),
            pl.BlockSpec((2 * D, D), lambda m: (0, 0)),
            pl.BlockSpec((1, D), lambda m: (0, 0)),
            pl.BlockSpec((D, D), lambda m: (0, 0)),
            pl.BlockSpec((1, D), lambda m: (0, 0)),
            pl.BlockSpec((1, D), lambda m: (0, 0)),
            pl.BlockSpec((1, D), lambda m: (0, 0)),
        ],
        out_specs=pl.BlockSpec((tm, D), lambda m: (m, 0)),
        out_shape=jax.ShapeDtypeStruct((N, D), jnp.float32),
    )(h, agg, w1, b1.reshape(1, D), w2, b2.reshape(1, D),
      g.reshape(1, D), bb.reshape(1, D))


def _out_body(q_ref, w1_ref, b1_ref, w2_ref, o_ref):
    t = jnp.maximum(
        jnp.dot(q_ref[...], w1_ref[...], preferred_element_type=jnp.float32)
        + b1_ref[...], 0.0)
    o_ref[...] = jnp.sum(t * w2_ref[...], axis=1, keepdims=True)


def _out_mlp(q, w1, b1, w2, b2):
    tm = 1024
    s = pl.pallas_call(
        _out_body,
        grid=(NQ // tm,),
        in_specs=[
            pl.BlockSpec((tm, D), lambda m: (m, 0)),
            pl.BlockSpec((D, D), lambda m: (0, 0)),
            pl.BlockSpec((1, D), lambda m: (0, 0)),
            pl.BlockSpec((1, D), lambda m: (0, 0)),
        ],
        out_specs=pl.BlockSpec((tm, 1), lambda m: (m, 0)),
        out_shape=jax.ShapeDtypeStruct((NQ, 1), jnp.float32),
    )(q, w1, b1.reshape(1, D), w2.reshape(1, D))
    return s + b2


@functools.lru_cache(maxsize=None)
def _sc_aggregate_fn():
    mesh = plsc.VectorSubcoreMesh(core_axis_name="c", subcore_axis_name="s")

    @functools.partial(
        pl.kernel,
        mesh=mesh,
        out_type=jax.ShapeDtypeStruct((_NC, _NACC, DH), jnp.float32),
        scratch_types=[
            pltpu.VMEM((2, _ICH, _EB), jnp.int32),
            pltpu.VMEM((2, _ICH, _EB), jnp.int32),
            pltpu.VMEM((2, _EB, DH), jnp.float32),
            pltpu.VMEM_SHARED((_NACC, DH), jnp.float32),
            pltpu.SemaphoreType.DMA,
            pltpu.SemaphoreType.DMA,
            pltpu.SemaphoreType.DMA,
            pltpu.SemaphoreType.DMA,
            pltpu.SemaphoreType.DMA,
        ],
    )
    def aggr(msg_hbm, src_hbm, dst_hbm, zero_hbm, out_hbm,
             src_v, dst_v, rows_v, acc_sh,
             gsem0, gsem1, ssem0, ssem1, isem):
        c = lax.axis_index("c")
        s = lax.axis_index("s")
        npt = _NACC // _NS
        pltpu.sync_copy(zero_hbm.at[pl.ds(s * npt, npt)],
                        acc_sh.at[pl.ds(s * npt, npt)])
        plsc.subcore_barrier()
        msg_c = msg_hbm.at[c]
        base = s * _RPT
        gsems = (gsem0, gsem1)
        ssems = (ssem0, ssem1)

        pltpu.make_async_copy(src_hbm.at[pl.ds(base, _ICH)],
                              src_v.at[0], isem).start()
        pltpu.make_async_copy(dst_hbm.at[pl.ds(base, _ICH)],
                              dst_v.at[0], isem).start()

        def chunk_body(ci, carry):
            slot = lax.rem(ci, 2)
            nslot = 1 - slot
            r0 = base + ci * _ICH
            pltpu.make_async_copy(src_hbm.at[pl.ds(r0, _ICH)],
                                  src_v.at[slot], isem).wait()
            pltpu.make_async_copy(dst_hbm.at[pl.ds(r0, _ICH)],
                                  dst_v.at[slot], isem).wait()

            @pl.when(ci + 1 < _NICH)
            def _():
                r1 = base + (ci + 1) * _ICH
                pltpu.make_async_copy(src_hbm.at[pl.ds(r1, _ICH)],
                                      src_v.at[nslot], isem).start()
                pltpu.make_async_copy(dst_hbm.at[pl.ds(r1, _ICH)],
                                      dst_v.at[nslot], isem).start()

            sv = src_v.at[slot]
            dv = dst_v.at[slot]
            pltpu.make_async_copy(msg_c.at[sv.at[0]], rows_v.at[0],
                                  gsems[0]).start()
            for r in range(_ICH):
                b = r % 2
                nb = 1 - b
                if r + 1 < _ICH:
                    if r >= 1:
                        pltpu.make_async_copy(rows_v.at[nb],
                                              acc_sh.at[dv.at[r - 1]],
                                              ssems[nb]).wait()
                    pltpu.make_async_copy(msg_c.at[sv.at[r + 1]],
                                          rows_v.at[nb], gsems[nb]).start()
                pltpu.make_async_copy(msg_c.at[sv.at[r]], rows_v.at[b],
                                      gsems[b]).wait()
                pltpu.make_async_copy(rows_v.at[b], acc_sh.at[dv.at[r]],
                                      ssems[b]).start(add=True)
            pltpu.make_async_copy(rows_v.at[0], acc_sh.at[dv.at[_ICH - 2]],
                                  ssems[0]).wait()
            pltpu.make_async_copy(rows_v.at[1], acc_sh.at[dv.at[_ICH - 1]],
                                  ssems[1]).wait()
            return carry

        lax.fori_loop(0, _NICH, chunk_body, 0)
        plsc.subcore_barrier()
        pltpu.sync_copy(acc_sh.at[pl.ds(s * npt, npt)],
                        out_hbm.at[c, pl.ds(s * npt, npt)])

    return aggr


def kernel(x, edge_index, n_qubits, in_W, in_b, msg_W1, msg_b1, msg_W2, msg_b2,
           upd_W1, upd_b1, upd_W2, upd_b2, ln_g, ln_b,
           out_W1, out_b1, out_W2, out_b2):
    pad = _EPAD - E
    src_r = jnp.concatenate(
        [edge_index[0], jnp.zeros((pad,), edge_index.dtype)]).reshape(_EROWS, _EB)
    dst_r = jnp.concatenate(
        [edge_index[1], jnp.full((pad,), N, edge_index.dtype)]).reshape(_EROWS, _EB)
    zeros = jnp.zeros((_NACC, DH), jnp.float32)
    aggr_fn = _sc_aggregate_fn()

    h = _in_mlp(x, in_W, in_b)
    for i in range(L):
        msg2 = _msg_mlp(h, msg_W1[i], msg_b1[i], msg_W2[i], msg_b2[i])
        agg2 = aggr_fn(msg2, src_r, dst_r, zeros)
        h = _upd_mlp(h, agg2, upd_W1[i], upd_b1[i], upd_W2[i], upd_b2[i],
                     ln_g[i], ln_b[i])
    q = lax.dynamic_slice_in_dim(h, n_qubits - NQ, NQ, axis=0)
    return _out_mlp(q, out_W1, out_b1, out_W2, out_b2)
